# speedup vs baseline: 3.0657x; 3.0657x over previous
import functools

import jax
import jax.numpy as jnp
from jax import lax
from jax.experimental import pallas as pl
from jax.experimental.pallas import tpu as pltpu
from jax.experimental.pallas import tpu_sc as plsc

NUM_CORES = 2
NUM_SUBCORES = 16
NW = NUM_CORES * NUM_SUBCORES


@functools.partial(jax.jit, static_argnums=(1, 2))
def _posemb_sc(table, b, t):
    d = table.shape[1]
    rows_per_w = t // NW
    chunk = min(rows_per_w, 64)
    n_chunks = rows_per_w // chunk

    mesh = plsc.VectorSubcoreMesh(core_axis_name="c", subcore_axis_name="s")

    @functools.partial(
        pl.kernel,
        mesh=mesh,
        out_type=jax.ShapeDtypeStruct((b * t, d), jnp.float32),
        scratch_types=[
            pltpu.VMEM((chunk, d), jnp.float32),
        ],
    )
    def body(table_hbm, out_hbm, buf):
        wid = lax.axis_index("s") * NUM_CORES + lax.axis_index("c")
        base = wid * rows_per_w
        for c in range(n_chunks):
            r0 = base + c * chunk
            pltpu.sync_copy(table_hbm.at[pl.ds(r0, chunk)], buf)
            for bi in range(b):
                pltpu.sync_copy(buf, out_hbm.at[pl.ds(bi * t + r0, chunk)])

    return body(table)


def kernel(x, positional_emb):
    b, t = x.shape
    assert t % NW == 0
    out = _posemb_sc(positional_emb, b, t)
    return out.reshape(b, t, positional_emb.shape[1])
